# Initial kernel scaffold; baseline (speedup 1.0000x reference)
import jax
import jax.numpy as jnp
from jax.experimental import pallas as pl


def kernel(x, weight):
    raise NotImplementedError("write your pallas kernel here")


# speedup vs baseline: 1.5647x; 1.5647x over previous
import functools

import jax
import jax.numpy as jnp
from jax import lax
from jax.experimental import pallas as pl
from jax.experimental.pallas import tpu as pltpu
from jax.experimental.pallas import tpu_sc as plsc

ROWS = 16384
COLS = 26
DIM = 32
N = ROWS * COLS

_INFO = plsc.get_sparse_core_info()
NC = _INFO.num_cores
NS = _INFO.num_subcores
NW = NC * NS

PER_W = N // NW
IDXV = 128
GROUP = 8
CHUNK = IDXV * GROUP
NCHUNK = PER_W // CHUNK
NIDX = PER_W // IDXV

_mesh = plsc.VectorSubcoreMesh(core_axis_name="c", subcore_axis_name="s")


@functools.partial(
    pl.kernel,
    mesh=_mesh,
    out_type=jax.ShapeDtypeStruct((N, DIM), jnp.float32),
    compiler_params=pltpu.CompilerParams(use_tc_tiling_on_sc=False),
    scratch_types=[
        pltpu.VMEM((NIDX, IDXV), jnp.int32),
        pltpu.VMEM((CHUNK, DIM), jnp.float32),
        pltpu.SemaphoreType.DMA,
    ],
)
def _emb_lookup(idx_hbm, table_hbm, out_hbm, idx_v, rows_v, sem):
    wid = lax.axis_index("s") * NC + lax.axis_index("c")
    base = wid * PER_W
    pltpu.sync_copy(idx_hbm.at[wid], idx_v)

    def chunk_body(c, _):
        copies = []
        for j in range(GROUP):
            copies.append(
                pltpu.async_copy(
                    table_hbm.at[idx_v.at[c * GROUP + j]],
                    rows_v.at[pl.ds(j * IDXV, IDXV)],
                    sem,
                )
            )
        for cp in copies:
            cp.wait()
        pltpu.sync_copy(rows_v, out_hbm.at[pl.ds(base + c * CHUNK, CHUNK)])
        return ()

    lax.fori_loop(0, NCHUNK, chunk_body, (), unroll=False)


def kernel(x, weight):
    idx = x.reshape(NW, NIDX, IDXV)
    out = _emb_lookup(idx, weight)
    return out.reshape(ROWS, COLS, DIM)


# speedup vs baseline: 1.5737x; 1.0058x over previous
import functools

import jax
import jax.numpy as jnp
from jax import lax
from jax.experimental import pallas as pl
from jax.experimental.pallas import tpu as pltpu
from jax.experimental.pallas import tpu_sc as plsc

ROWS = 16384
COLS = 26
DIM = 32
N = ROWS * COLS

_INFO = plsc.get_sparse_core_info()
NC = _INFO.num_cores
NS = _INFO.num_subcores
NW = NC * NS

PER_W = N // NW
IDXV = 128
GROUP = 8
CHUNK = IDXV * GROUP
NCHUNK = PER_W // CHUNK
NIDX = PER_W // IDXV
NBUF = 3

_mesh = plsc.VectorSubcoreMesh(core_axis_name="c", subcore_axis_name="s")


@functools.partial(
    pl.kernel,
    mesh=_mesh,
    out_type=jax.ShapeDtypeStruct((N, DIM), jnp.float32),
    compiler_params=pltpu.CompilerParams(use_tc_tiling_on_sc=False),
    scratch_types=[
        pltpu.VMEM((NIDX, IDXV), jnp.int32),
        pltpu.VMEM((NBUF, CHUNK, DIM), jnp.float32),
        pltpu.SemaphoreType.DMA((NBUF,)),
        pltpu.SemaphoreType.DMA((NBUF,)),
    ],
)
def _emb_lookup(idx_hbm, table_hbm, out_hbm, idx_v, rows_v, sem_g, sem_w):
    wid = lax.axis_index("s") * NC + lax.axis_index("c")
    base = wid * PER_W
    pltpu.sync_copy(idx_hbm.at[wid], idx_v)

    def fire_gathers(c, b):
        for j in range(GROUP):
            pltpu.async_copy(
                table_hbm.at[idx_v.at[c * GROUP + j]],
                rows_v.at[b, pl.ds(j * IDXV, IDXV)],
                sem_g.at[b],
            )

    def wait_gathers(b):
        pltpu.make_async_copy(
            out_hbm.at[pl.ds(0, CHUNK)], rows_v.at[b], sem_g.at[b]
        ).wait()

    def fire_writeback(c, b):
        pltpu.async_copy(
            rows_v.at[b], out_hbm.at[pl.ds(base + c * CHUNK, CHUNK)], sem_w.at[b]
        )

    def wait_writeback(b):
        pltpu.make_async_copy(
            rows_v.at[b], out_hbm.at[pl.ds(0, CHUNK)], sem_w.at[b]
        ).wait()

    for b in range(NBUF):
        fire_gathers(b, b)

    def chunk_body(c, _):
        b = lax.rem(c, NBUF)
        wait_gathers(b)
        fire_writeback(c, b)

        @pl.when(jnp.logical_and(c >= 1, c + 2 < NCHUNK))
        def _():
            bn = lax.rem(c + 2, NBUF)
            wait_writeback(bn)
            fire_gathers(c + 2, bn)

        return ()

    lax.fori_loop(0, NCHUNK, chunk_body, (), unroll=False)

    for b in range(NBUF):
        wait_writeback(b)


def kernel(x, weight):
    idx = x.reshape(NW, NIDX, IDXV)
    out = _emb_lookup(idx, weight)
    return out.reshape(ROWS, COLS, DIM)
